# baseline (speedup 1.0000x reference)
import functools

import jax
import jax.numpy as jnp
from jax import lax
from jax.experimental import pallas as pl
from jax.experimental.pallas import tpu as pltpu
from jax.experimental.pallas import tpu_sc as plsc

VOCAB = 1000000
BATCH = 16384
HIST = 200

_info = plsc.get_sparse_core_info()
NC = _info.num_cores
NS = _info.num_subcores
NW = NC * NS
ROWS_PER_TILE = BATCH // NW
NCHUNK = 8
CROWS = ROWS_PER_TILE // NCHUNK
STAGE_HOP = 5208
STAGE_NHOP = 12
STAGE = STAGE_HOP * STAGE_NHOP
STAGE_TAIL = VOCAB - 16 * STAGE

_mesh = plsc.VectorSubcoreMesh(core_axis_name="c", subcore_axis_name="s")


@functools.partial(
    pl.kernel,
    mesh=_mesh,
    out_type=jax.ShapeDtypeStruct((BATCH, HIST), jnp.float32),
    compiler_params=pltpu.CompilerParams(use_tc_tiling_on_sc=False),
    scratch_types=[
        pltpu.VMEM((CROWS, HIST), jnp.int32),
        pltpu.VMEM((CROWS, HIST), jnp.int32),
        pltpu.VMEM((CROWS, HIST), jnp.float32),
        pltpu.VMEM((CROWS, HIST), jnp.float32),
        pltpu.VMEM_SHARED((1, VOCAB), jnp.float32),
        pltpu.VMEM((1, STAGE_HOP), jnp.float32),
        pltpu.VMEM((1, STAGE_HOP), jnp.float32),
        pltpu.SemaphoreType.DMA,
        pltpu.SemaphoreType.DMA,
        pltpu.SemaphoreType.DMA,
        pltpu.SemaphoreType.DMA,
        pltpu.SemaphoreType.DMA,
        pltpu.SemaphoreType.DMA,
        pltpu.SemaphoreType.DMA,
    ],
)
def _gather_kernel(ids_hbm, table_hbm, out_hbm, idx0, idx1, vals0, vals1,
                   table_sh, bounce0, bounce1,
                   isem0, isem1, gsem0, gsem1, ssem0, ssem1, stsem):
    sid = lax.axis_index("s")
    wid = sid * NC + lax.axis_index("c")
    base = wid * ROWS_PER_TILE
    idx = (idx0, idx1)
    vals = (vals0, vals1)
    bounce = (bounce0, bounce1)
    isem = (isem0, isem1)
    gsem = (gsem0, gsem1)
    ssem = (ssem0, ssem1)

    def load(i):
        pltpu.async_copy(
            ids_hbm.at[pl.ds(base + i * CROWS, CROWS)], idx[i % 2],
            isem[i % 2])

    def wait_load(i):
        pltpu.make_async_copy(
            ids_hbm.at[pl.ds(base + i * CROWS, CROWS)], idx[i % 2],
            isem[i % 2]).wait()

    def store(i):
        pltpu.async_copy(
            vals[i % 2], out_hbm.at[pl.ds(base + i * CROWS, CROWS)],
            ssem[i % 2])

    def wait_store(i):
        pltpu.make_async_copy(
            vals[i % 2], out_hbm.at[pl.ds(base + i * CROWS, CROWS)],
            ssem[i % 2]).wait()

    def gather(i):
        b = i % 2

        def enq(r, _):
            pltpu.async_copy(table_sh.at[idx[b].at[pl.ds(r, 1), :]],
                             vals[b].at[pl.ds(r, 1), :], gsem[b])
            return 0

        lax.fori_loop(0, CROWS, enq, 0, unroll=8)

    def wait_gather(i):
        b = i % 2

        def drain(r, _):
            pltpu.make_async_copy(table_sh.at[idx[b].at[pl.ds(r, 1), :]],
                                  vals[b].at[pl.ds(r, 1), :], gsem[b]).wait()
            return 0

        lax.fori_loop(0, CROWS, drain, 0, unroll=8)

    load(0)
    load(1)

    stage = sid * STAGE
    for h in range(STAGE_NHOP):
        off = stage + h * STAGE_HOP
        b = bounce[h % 2]
        if h >= 2:
            pltpu.make_async_copy(
                b,
                table_sh.at[:, pl.ds(stage + (h - 2) * STAGE_HOP, STAGE_HOP)],
                stsem).wait()
        pltpu.sync_copy(table_hbm.at[pl.ds(off, STAGE_HOP)], b.at[0])
        pltpu.async_copy(b, table_sh.at[:, pl.ds(off, STAGE_HOP)], stsem)
    for h in range(STAGE_NHOP - 2, STAGE_NHOP):
        off = stage + h * STAGE_HOP
        pltpu.make_async_copy(
            bounce[h % 2], table_sh.at[:, pl.ds(off, STAGE_HOP)], stsem).wait()

    @pl.when(sid == 0)
    def _stage_tail():
        pltpu.sync_copy(table_hbm.at[pl.ds(NS * STAGE, STAGE_TAIL)],
                        bounce0.at[0, pl.ds(0, STAGE_TAIL)])
        pltpu.sync_copy(bounce0.at[:, pl.ds(0, STAGE_TAIL)],
                        table_sh.at[:, pl.ds(NS * STAGE, STAGE_TAIL)])

    plsc.subcore_barrier()

    wait_load(0)
    gather(0)

    for i in range(1, NCHUNK):
        wait_load(i)
        if i >= 2:
            wait_store(i - 2)
        gather(i)
        wait_gather(i - 1)
        store(i - 1)
        if i + 1 < NCHUNK:
            load(i + 1)

    wait_gather(NCHUNK - 1)
    store(NCHUNK - 1)
    wait_store(NCHUNK - 2)
    wait_store(NCHUNK - 1)


def kernel(input_ids, mapping_table):
    return _gather_kernel(input_ids, mapping_table)


# speedup vs baseline: 1.0315x; 1.0315x over previous
import functools

import jax
import jax.numpy as jnp
from jax import lax
from jax.experimental import pallas as pl
from jax.experimental.pallas import tpu as pltpu
from jax.experimental.pallas import tpu_sc as plsc

VOCAB = 1000000
BATCH = 16384
HIST = 200
TOTAL = BATCH * HIST

_info = plsc.get_sparse_core_info()
NC = _info.num_cores
NS = _info.num_subcores
NW = NC * NS
PER_TILE = TOTAL // NW
NCHUNK = 8
CHUNK = PER_TILE // NCHUNK
STAGE_HOP = 10416
STAGE_NHOP = 6
STAGE = STAGE_HOP * STAGE_NHOP
STAGE_TAIL = VOCAB - 16 * STAGE

_mesh = plsc.VectorSubcoreMesh(core_axis_name="c", subcore_axis_name="s")


@functools.partial(
    pl.kernel,
    mesh=_mesh,
    out_type=jax.ShapeDtypeStruct((TOTAL,), jnp.float32),
    scratch_types=[
        pltpu.VMEM((CHUNK,), jnp.int32),
        pltpu.VMEM((CHUNK,), jnp.int32),
        pltpu.VMEM((CHUNK,), jnp.float32),
        pltpu.VMEM((CHUNK,), jnp.float32),
        pltpu.VMEM_SHARED((VOCAB,), jnp.float32),
        pltpu.VMEM((STAGE_HOP,), jnp.float32),
        pltpu.SemaphoreType.DMA,
        pltpu.SemaphoreType.DMA,
        pltpu.SemaphoreType.DMA,
        pltpu.SemaphoreType.DMA,
        pltpu.SemaphoreType.DMA,
    ],
)
def _gather_kernel(ids_hbm, table_hbm, out_hbm, idx0, idx1, vals0, vals1,
                   table_sh, bounce, isem0, isem1, gsem, ssem0, ssem1):
    sid = lax.axis_index("s")
    wid = sid * NC + lax.axis_index("c")
    base = wid * PER_TILE
    idx = (idx0, idx1)
    vals = (vals0, vals1)
    isem = (isem0, isem1)
    ssem = (ssem0, ssem1)

    stage = sid * STAGE
    for h in range(STAGE_NHOP):
        off = stage + h * STAGE_HOP
        pltpu.sync_copy(table_hbm.at[pl.ds(off, STAGE_HOP)], bounce)
        pltpu.sync_copy(bounce, table_sh.at[pl.ds(off, STAGE_HOP)])

    @pl.when(sid == 0)
    def _stage_tail():
        pltpu.sync_copy(table_hbm.at[pl.ds(NS * STAGE, STAGE_TAIL)],
                        bounce.at[pl.ds(0, STAGE_TAIL)])
        pltpu.sync_copy(bounce.at[pl.ds(0, STAGE_TAIL)],
                        table_sh.at[pl.ds(NS * STAGE, STAGE_TAIL)])

    plsc.subcore_barrier()

    for b in range(2):
        pltpu.async_copy(
            ids_hbm.at[pl.ds(base + b * CHUNK, CHUNK)], idx[b], isem[b])

    for i in range(NCHUNK):
        b = i % 2
        pltpu.make_async_copy(
            ids_hbm.at[pl.ds(base + i * CHUNK, CHUNK)], idx[b],
            isem[b]).wait()
        if i >= 2:
            pltpu.make_async_copy(
                vals[b], out_hbm.at[pl.ds(base + (i - 2) * CHUNK, CHUNK)],
                ssem[b]).wait()
        pltpu.async_copy(table_sh.at[idx[b]], vals[b], gsem).wait()
        pltpu.async_copy(
            vals[b], out_hbm.at[pl.ds(base + i * CHUNK, CHUNK)], ssem[b])
        if i + 2 < NCHUNK:
            pltpu.async_copy(
                ids_hbm.at[pl.ds(base + (i + 2) * CHUNK, CHUNK)], idx[b],
                isem[b])

    for i in range(NCHUNK - 2, NCHUNK):
        b = i % 2
        pltpu.make_async_copy(
            vals[b], out_hbm.at[pl.ds(base + i * CHUNK, CHUNK)],
            ssem[b]).wait()


def kernel(input_ids, mapping_table):
    flat_ids = input_ids.reshape(TOTAL)
    out = _gather_kernel(flat_ids, mapping_table)
    return out.reshape(BATCH, HIST)
